# baseline (speedup 1.0000x reference)
import functools

import jax
import jax.numpy as jnp
from jax import lax
from jax.experimental import pallas as pl
from jax.experimental.pallas import tpu as pltpu
from jax.experimental.pallas import tpu_sc as plsc

E = 8
K = 2
D = 768
H = 768
S = 2048
NPAIR = S * K
TMC = 256
NP = NPAIR + E * TMC
NT = NPAIR // TMC + E
NC = 2
NS = 16
NW = NC * NS
BP = NPAIR // NW
TW = S // NW
SUB = 32


TS = 256


def _router_kernel(x_ref, wr1_ref, br1_ref, wg_ref,
                   oh1_ref, oh2_ref, gp1_ref, gp2_ref):
    xb = x_ref[...]
    h = lax.dot_general(xb, wr1_ref[...], (((1,), (1,)), ((), ())),
                        preferred_element_type=jnp.float32)
    h = jnp.tanh(h + br1_ref[...])
    lt = lax.dot_general(wg_ref[...], h, (((1,), (1,)), ((), ())),
                         preferred_element_type=jnp.float32)
    m = jnp.max(lt, axis=0, keepdims=True)
    p = jnp.exp(lt - m)
    p = p / jnp.sum(p, axis=0, keepdims=True)
    e_iota = lax.broadcasted_iota(jnp.int32, p.shape, 0)
    m1 = jnp.max(p, axis=0, keepdims=True)
    i1 = jnp.min(jnp.where(p == m1, e_iota, E), axis=0, keepdims=True)
    p_rest = jnp.where(e_iota == i1, -jnp.inf, p)
    m2 = jnp.max(p_rest, axis=0, keepdims=True)
    i2 = jnp.min(jnp.where(p_rest == m2, e_iota, E), axis=0, keepdims=True)
    denom = m1 + m2 + 1e-6
    g1 = m1 / denom
    g2 = m2 / denom
    pack = jnp.concatenate(
        [i1.astype(jnp.float32), i2.astype(jnp.float32), g1, g2,
         jnp.zeros((4, TS), jnp.float32)], axis=0)
    packT = lax.transpose(pack, (1, 0))
    i1c = packT[:, 0:1].astype(jnp.int32)
    i2c = packT[:, 1:2].astype(jnp.int32)
    e8 = lax.broadcasted_iota(jnp.int32, (TS, E), 1)
    oh1_ref[...] = (e8 == i1c).astype(jnp.float32)
    oh2_ref[...] = (e8 == i2c).astype(jnp.float32)
    gp1_ref[...] = jnp.broadcast_to(packT[:, 2:3], (TS, 128))
    gp2_ref[...] = jnp.broadcast_to(packT[:, 3:4], (TS, 128))


def _router_call(x2d, Wr1, br1_2d, Wg, interpret=False):
    return pl.pallas_call(
        _router_kernel,
        grid=(S // TS,),
        in_specs=[
            pl.BlockSpec((TS, D), lambda i: (i, 0)),
            pl.BlockSpec((D, D), lambda i: (0, 0)),
            pl.BlockSpec((1, D), lambda i: (0, 0)),
            pl.BlockSpec((E, D), lambda i: (0, 0)),
        ],
        out_specs=[
            pl.BlockSpec((TS, E), lambda i: (i, 0)),
            pl.BlockSpec((TS, E), lambda i: (i, 0)),
            pl.BlockSpec((TS, 128), lambda i: (i, 0)),
            pl.BlockSpec((TS, 128), lambda i: (i, 0)),
        ],
        out_shape=[
            jax.ShapeDtypeStruct((S, E), jnp.float32),
            jax.ShapeDtypeStruct((S, E), jnp.float32),
            jax.ShapeDtypeStruct((S, 128), jnp.float32),
            jax.ShapeDtypeStruct((S, 128), jnp.float32),
        ],
        compiler_params=pltpu.CompilerParams(
            dimension_semantics=("arbitrary",),
            vmem_limit_bytes=60 * 1024 * 1024,
        ),
        interpret=interpret,
    )(x2d, Wr1, br1_2d, Wg)


def _rank_kernel(oh1_ref, oh2_ref, dpos_ref, cnt_ref):
    r_i = lax.broadcasted_iota(jnp.int32, (256, 256), 0)
    c_i = lax.broadcasted_iota(jnp.int32, (256, 256), 1)
    tri = (c_i < r_i).astype(jnp.float32)

    nblk = NPAIR // 256

    def ohblk(b):
        ref = oh1_ref if b < nblk // 2 else oh2_ref
        o = (b % (nblk // 2)) * 256
        return ref[o:o + 256, :]

    t_rows = []
    rank_blocks = []
    for b in range(nblk):
        ohb = ohblk(b)
        rb = lax.dot_general(tri, ohb, (((1,), (0,)), ((), ())),
                             preferred_element_type=jnp.float32)
        rank_blocks.append(rb)
        t_rows.append(jnp.sum(ohb, axis=0, keepdims=True))
    t_mat = jnp.concatenate(t_rows, axis=0)

    r16 = lax.broadcasted_iota(jnp.int32, (nblk, nblk), 0)
    c16 = lax.broadcasted_iota(jnp.int32, (nblk, nblk), 1)
    tri16 = (r16 < c16).astype(jnp.float32)
    bo = lax.dot_general(tri16, t_mat, (((0,), (0,)), ((), ())),
                         preferred_element_type=jnp.float32)

    counts = jnp.sum(t_mat, axis=0, keepdims=True)
    ci = counts.astype(jnp.int32)
    pci = ((ci + (TMC - 1)) // TMC) * TMC
    pcf = pci.astype(jnp.float32)
    r8 = lax.broadcasted_iota(jnp.int32, (E, E), 0)
    c8 = lax.broadcasted_iota(jnp.int32, (E, E), 1)
    tri8 = (r8 < c8).astype(jnp.float32)
    eo = lax.dot_general(pcf, tri8, (((1,), (0,)), ((), ())),
                         preferred_element_type=jnp.float32)

    dpos_parts = []
    for b in range(nblk):
        ohb = ohblk(b)
        base = rank_blocks[b] + bo[b:b + 1, :] + eo
        dpos_parts.append(jnp.sum(base * ohb, axis=1, keepdims=True))
    dpos = jnp.concatenate(dpos_parts, axis=0)
    dpos_ref[...] = dpos.astype(jnp.int32)
    cnt_ref[...] = ci


def _rank_call(oh1, oh2, interpret=False):
    return pl.pallas_call(
        _rank_kernel,
        in_specs=[
            pl.BlockSpec((S, E), lambda: (0, 0)),
            pl.BlockSpec((S, E), lambda: (0, 0)),
        ],
        out_specs=[
            pl.BlockSpec((NPAIR, 1), lambda: (0, 0)),
            pl.BlockSpec((1, E), lambda: (0, 0)),
        ],
        out_shape=[
            jax.ShapeDtypeStruct((NPAIR, 1), jnp.int32),
            jax.ShapeDtypeStruct((1, E), jnp.int32),
        ],
        interpret=interpret,
    )(oh1, oh2)


def _scatter_kernel(dpos_hbm, x_hbm, gp1_hbm, gp2_hbm, xs_hbm, gs_hbm,
                    idx_v, rows_v, g_v, sem1, sem2):
    wid = lax.axis_index("s") * NC + lax.axis_index("c")
    pbase = wid * BP
    tokbase = lax.rem(pbase, S)
    pltpu.sync_copy(dpos_hbm.at[pl.ds(pbase, BP)], idx_v)
    pltpu.sync_copy(x_hbm.at[pl.ds(tokbase, BP)], rows_v)

    @pl.when(pbase < S)
    def _():
        pltpu.sync_copy(gp1_hbm.at[pl.ds(tokbase, BP)], g_v)

    @pl.when(pbase >= S)
    def _():
        pltpu.sync_copy(gp2_hbm.at[pl.ds(tokbase, BP)], g_v)
    cp1 = pltpu.async_copy(rows_v, xs_hbm.at[idx_v], sem1)
    cp2 = pltpu.async_copy(g_v, gs_hbm.at[idx_v], sem2)
    cp1.wait()
    cp2.wait()


@functools.lru_cache(maxsize=1)
def _scatter_call_builder():
    return pl.kernel(
        _scatter_kernel,
        mesh=plsc.VectorSubcoreMesh(core_axis_name="c", subcore_axis_name="s"),
        out_type=[
            jax.ShapeDtypeStruct((NP, D), jnp.float32),
            jax.ShapeDtypeStruct((NP, 128), jnp.float32),
        ],
        scratch_types=[
            pltpu.VMEM((BP,), jnp.int32),
            pltpu.VMEM((BP, D), jnp.float32),
            pltpu.VMEM((BP, 128), jnp.float32),
            pltpu.SemaphoreType.DMA,
            pltpu.SemaphoreType.DMA,
        ],
    )


def _scatter_call(dpos_hbm, x_hbm, gp1_hbm, gp2_hbm):
    return _scatter_call_builder()(dpos_hbm, x_hbm, gp1_hbm, gp2_hbm)


def _gmm_kernel(te_ref, xs_ref, gs_ref, win_ref, wout_ref, out_ref):
    e = te_ref[pl.program_id(0)]
    hh = lax.dot_general(xs_ref[...], win_ref[e], (((1,), (0,)), ((), ())),
                         preferred_element_type=jnp.float32)
    hh = jnp.maximum(hh, 0.0) * gs_ref[:, 0:1]
    oo = lax.dot_general(hh, wout_ref[e], (((1,), (0,)), ((), ())),
                         preferred_element_type=jnp.float32)
    out_ref[...] = oo


def _gmm_call(te, xs, gs, We_in, We_out, interpret=False):
    return pl.pallas_call(
        _gmm_kernel,
        grid_spec=pltpu.PrefetchScalarGridSpec(
            num_scalar_prefetch=1,
            grid=(NT,),
            in_specs=[
                pl.BlockSpec((TMC, D), lambda t, te_ref: (t, 0)),
                pl.BlockSpec((TMC, 128), lambda t, te_ref: (t, 0)),
                pl.BlockSpec((E, D, H), lambda t, te_ref: (0, 0, 0)),
                pl.BlockSpec((E, H, D), lambda t, te_ref: (0, 0, 0)),
            ],
            out_specs=pl.BlockSpec((TMC, D), lambda t, te_ref: (t, 0)),
        ),
        out_shape=jax.ShapeDtypeStruct((NP, D), jnp.float32),
        compiler_params=pltpu.CompilerParams(
            dimension_semantics=("arbitrary",),
            vmem_limit_bytes=60 * 1024 * 1024,
        ),
        interpret=interpret,
    )(te, xs, gs, We_in, We_out)


def _combine_kernel(dp0_hbm, dp1_hbm, outs_hbm, y_hbm,
                    p0_v, p1_v, r0_v, r1_v, sem0, sem1):
    wid = lax.axis_index("s") * NC + lax.axis_index("c")
    for sub in range(TW // SUB):
        base = wid * TW + sub * SUB
        pltpu.sync_copy(dp0_hbm.at[pl.ds(base, SUB)], p0_v)
        pltpu.sync_copy(dp1_hbm.at[pl.ds(base, SUB)], p1_v)
        cp0 = pltpu.async_copy(outs_hbm.at[p0_v], r0_v, sem0)
        cp1 = pltpu.async_copy(outs_hbm.at[p1_v], r1_v, sem1)
        cp0.wait()
        cp1.wait()

        def _row(r, carry):
            for j in range(D // 16):
                sl = pl.ds(j * 16, 16)
                r0_v[r, sl] = r0_v[r, sl] + r1_v[r, sl]
            return carry

        lax.fori_loop(0, SUB, _row, 0)
        pltpu.sync_copy(r0_v, y_hbm.at[pl.ds(base, SUB)])


@functools.lru_cache(maxsize=1)
def _combine_call_builder():
    return pl.kernel(
        _combine_kernel,
        mesh=plsc.VectorSubcoreMesh(core_axis_name="c", subcore_axis_name="s"),
        out_type=jax.ShapeDtypeStruct((S, D), jnp.float32),
        scratch_types=[
            pltpu.VMEM((SUB,), jnp.int32),
            pltpu.VMEM((SUB,), jnp.int32),
            pltpu.VMEM((SUB, D), jnp.float32),
            pltpu.VMEM((SUB, D), jnp.float32),
            pltpu.SemaphoreType.DMA,
            pltpu.SemaphoreType.DMA,
        ],
    )


def _combine_call(dp0_hbm, dp1_hbm, outs_hbm):
    return _combine_call_builder()(dp0_hbm, dp1_hbm, outs_hbm)


def kernel(x, Wr1, br1, Wg, We_in, We_out):
    bsz, length, d = x.shape
    x2d = x.reshape(S, D)
    oh1, oh2, gp1, gp2 = _router_call(x2d, Wr1, br1.reshape(1, -1), Wg)
    dpos, counts = _rank_call(oh1, oh2)
    dposf = dpos.reshape(NPAIR)
    ci = counts.reshape(E)
    nt = (ci + TMC - 1) // TMC
    te = jnp.repeat(jnp.arange(E, dtype=jnp.int32), nt,
                    total_repeat_length=NT)
    xs, gs = _scatter_call(dposf, x2d, gp1, gp2)
    outs = _gmm_call(te, xs, gs, We_in, We_out)
    y2 = _combine_call(dposf[:S], dposf[S:], outs)
    loss = jnp.zeros((), dtype=jnp.float32)
    return y2.reshape(bsz, length, d), loss


# speedup vs baseline: 1.0491x; 1.0491x over previous
import functools

import jax
import jax.numpy as jnp
from jax import lax
from jax.experimental import pallas as pl
from jax.experimental.pallas import tpu as pltpu
from jax.experimental.pallas import tpu_sc as plsc

E = 8
K = 2
D = 768
H = 768
S = 2048
NPAIR = S * K
TMC = 256
NP = NPAIR + E * TMC
NT = NPAIR // TMC + E
NC = 2
NS = 16
NW = NC * NS
BP = NPAIR // NW
TW = S // NW
SUB = 32
TS = 256
NB = S // TS


def _router_kernel(x_ref, wr1_ref, br1_ref, wg_ref,
                   gp1_ref, gp2_ref, dpos_ref, te_ref,
                   oh_scr):
    i = pl.program_id(0)

    @pl.when(i < NB)
    def _router_step():
        xb = x_ref[...]
        h = lax.dot_general(xb, wr1_ref[...], (((1,), (1,)), ((), ())),
                            preferred_element_type=jnp.float32)
        h = jnp.tanh(h + br1_ref[...])
        logits = lax.dot_general(h, wg_ref[...], (((1,), (1,)), ((), ())),
                                 preferred_element_type=jnp.float32)
        lt = lax.transpose(logits, (1, 0))
        m = jnp.max(lt, axis=0, keepdims=True)
        p = jnp.exp(lt - m)
        p = p / jnp.sum(p, axis=0, keepdims=True)
        e_iota = lax.broadcasted_iota(jnp.int32, p.shape, 0)
        m1 = jnp.max(p, axis=0, keepdims=True)
        i1 = jnp.min(jnp.where(p == m1, e_iota, E), axis=0, keepdims=True)
        p_rest = jnp.where(e_iota == i1, -jnp.inf, p)
        m2 = jnp.max(p_rest, axis=0, keepdims=True)
        i2 = jnp.min(jnp.where(p_rest == m2, e_iota, E), axis=0, keepdims=True)
        denom = m1 + m2 + 1e-6
        g1 = m1 / denom
        g2 = m2 / denom
        pack = jnp.concatenate(
            [i1.astype(jnp.float32), i2.astype(jnp.float32), g1, g2,
             jnp.zeros((4, TS), jnp.float32)], axis=0)
        packT = lax.transpose(pack, (1, 0))
        i1c = packT[:, 0:1].astype(jnp.int32)
        i2c = packT[:, 1:2].astype(jnp.int32)
        e8 = lax.broadcasted_iota(jnp.int32, (TS, E), 1)
        oh_scr[pl.ds(i * TS, TS), :] = (e8 == i1c).astype(jnp.float32)
        oh_scr[pl.ds(S + i * TS, TS), :] = (e8 == i2c).astype(jnp.float32)
        gp1_ref[...] = jnp.broadcast_to(packT[:, 2:3], (TS, 128))
        gp2_ref[...] = jnp.broadcast_to(packT[:, 3:4], (TS, 128))

    @pl.when(i == NB)
    def _rank_step():
        r_i = lax.broadcasted_iota(jnp.int32, (256, 256), 0)
        c_i = lax.broadcasted_iota(jnp.int32, (256, 256), 1)
        tri = (c_i < r_i).astype(jnp.float32)

        nblk = NPAIR // 256
        t_rows = []
        rank_blocks = []
        for b in range(nblk):
            ohb = oh_scr[b * 256:(b + 1) * 256, :]
            rb = lax.dot_general(tri, ohb, (((1,), (0,)), ((), ())),
                                 preferred_element_type=jnp.float32)
            rank_blocks.append(rb)
            t_rows.append(jnp.sum(ohb, axis=0, keepdims=True))
        t_mat = jnp.concatenate(t_rows, axis=0)

        r16 = lax.broadcasted_iota(jnp.int32, (nblk, nblk), 0)
        c16 = lax.broadcasted_iota(jnp.int32, (nblk, nblk), 1)
        tri16 = (r16 < c16).astype(jnp.float32)
        bo = lax.dot_general(tri16, t_mat, (((0,), (0,)), ((), ())),
                             preferred_element_type=jnp.float32)

        counts = jnp.sum(t_mat, axis=0, keepdims=True)
        ci = counts.astype(jnp.int32)
        pci = ((ci + (TMC - 1)) // TMC) * TMC
        pcf = pci.astype(jnp.float32)
        r8 = lax.broadcasted_iota(jnp.int32, (E, E), 0)
        c8 = lax.broadcasted_iota(jnp.int32, (E, E), 1)
        tri8 = (r8 < c8).astype(jnp.float32)
        eo = lax.dot_general(pcf, tri8, (((1,), (0,)), ((), ())),
                             preferred_element_type=jnp.float32)

        dpos_parts = []
        for b in range(nblk):
            ohb = oh_scr[b * 256:(b + 1) * 256, :]
            base = rank_blocks[b] + bo[b:b + 1, :] + eo
            dpos_parts.append(jnp.sum(base * ohb, axis=1, keepdims=True))
        dpos = jnp.concatenate(dpos_parts, axis=0)
        dpos_ref[...] = dpos.astype(jnp.int32)

        ctoT = lax.transpose(eo * (1.0 / TMC), (1, 0))
        t_iota = lax.broadcasted_iota(jnp.int32, (E, NT), 1).astype(jnp.float32)
        ge = jnp.sum((t_iota >= ctoT).astype(jnp.int32), axis=0,
                     keepdims=True)
        te_ref[...] = ge - 1


def _router_call(x2d, Wr1, br1_2d, Wg, interpret=False):
    nmap = lambda i: (jnp.minimum(i, NB - 1), 0)
    zmap = lambda i: (0, 0)
    return pl.pallas_call(
        _router_kernel,
        grid=(NB + 1,),
        in_specs=[
            pl.BlockSpec((TS, D), nmap),
            pl.BlockSpec((D, D), zmap),
            pl.BlockSpec((1, D), zmap),
            pl.BlockSpec((E, D), zmap),
        ],
        out_specs=[
            pl.BlockSpec((TS, 128), nmap),
            pl.BlockSpec((TS, 128), nmap),
            pl.BlockSpec((NPAIR, 1), zmap),
            pl.BlockSpec((1, NT), zmap),
        ],
        out_shape=[
            jax.ShapeDtypeStruct((S, 128), jnp.float32),
            jax.ShapeDtypeStruct((S, 128), jnp.float32),
            jax.ShapeDtypeStruct((NPAIR, 1), jnp.int32),
            jax.ShapeDtypeStruct((1, NT), jnp.int32),
        ],
        scratch_shapes=[pltpu.VMEM((NPAIR, E), jnp.float32)],
        compiler_params=pltpu.CompilerParams(
            dimension_semantics=("arbitrary",),
            vmem_limit_bytes=60 * 1024 * 1024,
        ),
        interpret=interpret,
    )(x2d, Wr1, br1_2d, Wg)


def _scatter_kernel(dpos_hbm, x_hbm, gp1_hbm, gp2_hbm, xs_hbm, gs_hbm,
                    idx0_v, idx1_v, rows_v, g0_v, g1_v, sem1, sem2):
    wid = lax.axis_index("s") * NC + lax.axis_index("c")
    base = wid * TW
    pltpu.sync_copy(dpos_hbm.at[pl.ds(base, TW)], idx0_v)
    pltpu.sync_copy(dpos_hbm.at[pl.ds(S + base, TW)], idx1_v)
    pltpu.sync_copy(x_hbm.at[pl.ds(base, TW)], rows_v)
    pltpu.sync_copy(gp1_hbm.at[pl.ds(base, TW)], g0_v)
    pltpu.sync_copy(gp2_hbm.at[pl.ds(base, TW)], g1_v)
    cp1 = pltpu.async_copy(rows_v, xs_hbm.at[idx0_v], sem1)
    cp2 = pltpu.async_copy(rows_v, xs_hbm.at[idx1_v], sem1)
    cp3 = pltpu.async_copy(g0_v, gs_hbm.at[idx0_v], sem2)
    cp4 = pltpu.async_copy(g1_v, gs_hbm.at[idx1_v], sem2)
    cp1.wait()
    cp2.wait()
    cp3.wait()
    cp4.wait()


@functools.lru_cache(maxsize=1)
def _scatter_call_builder():
    return pl.kernel(
        _scatter_kernel,
        mesh=plsc.VectorSubcoreMesh(core_axis_name="c", subcore_axis_name="s"),
        out_type=[
            jax.ShapeDtypeStruct((NP, D), jnp.float32),
            jax.ShapeDtypeStruct((NP, 128), jnp.float32),
        ],
        scratch_types=[
            pltpu.VMEM((TW,), jnp.int32),
            pltpu.VMEM((TW,), jnp.int32),
            pltpu.VMEM((TW, D), jnp.float32),
            pltpu.VMEM((TW, 128), jnp.float32),
            pltpu.VMEM((TW, 128), jnp.float32),
            pltpu.SemaphoreType.DMA,
            pltpu.SemaphoreType.DMA,
        ],
    )


def _scatter_call(dpos_hbm, x_hbm, gp1_hbm, gp2_hbm):
    return _scatter_call_builder()(dpos_hbm, x_hbm, gp1_hbm, gp2_hbm)


def _gmm_kernel(te_ref, xs_ref, gs_ref, win_ref, wout_ref, out_ref):
    e = te_ref[pl.program_id(0)]
    hh = lax.dot_general(xs_ref[...], win_ref[e], (((1,), (0,)), ((), ())),
                         preferred_element_type=jnp.float32)
    hh = jnp.maximum(hh, 0.0) * gs_ref[:, 0:1]
    oo = lax.dot_general(hh, wout_ref[e], (((1,), (0,)), ((), ())),
                         preferred_element_type=jnp.float32)
    out_ref[...] = oo


def _gmm_call(te, xs, gs, We_in, We_out, interpret=False):
    return pl.pallas_call(
        _gmm_kernel,
        grid_spec=pltpu.PrefetchScalarGridSpec(
            num_scalar_prefetch=1,
            grid=(NT,),
            in_specs=[
                pl.BlockSpec((TMC, D), lambda t, te_ref: (t, 0)),
                pl.BlockSpec((TMC, 128), lambda t, te_ref: (t, 0)),
                pl.BlockSpec((E, D, H), lambda t, te_ref: (0, 0, 0)),
                pl.BlockSpec((E, H, D), lambda t, te_ref: (0, 0, 0)),
            ],
            out_specs=pl.BlockSpec((TMC, D), lambda t, te_ref: (t, 0)),
        ),
        out_shape=jax.ShapeDtypeStruct((NP, D), jnp.float32),
        compiler_params=pltpu.CompilerParams(
            dimension_semantics=("arbitrary",),
            vmem_limit_bytes=60 * 1024 * 1024,
        ),
        interpret=interpret,
    )(te, xs, gs, We_in, We_out)


def _combine_kernel(dpos_hbm, outs_hbm, y_hbm,
                    p0_v, p1_v, r0_v, r1_v, sem0, sem1):
    wid = lax.axis_index("s") * NC + lax.axis_index("c")
    for sub in range(TW // SUB):
        base = wid * TW + sub * SUB
        pltpu.sync_copy(dpos_hbm.at[pl.ds(base, SUB)], p0_v)
        pltpu.sync_copy(dpos_hbm.at[pl.ds(S + base, SUB)], p1_v)
        cp0 = pltpu.async_copy(outs_hbm.at[p0_v], r0_v, sem0)
        cp1 = pltpu.async_copy(outs_hbm.at[p1_v], r1_v, sem1)
        cp0.wait()
        cp1.wait()

        def _row(r, carry):
            for j in range(D // 16):
                sl = pl.ds(j * 16, 16)
                r0_v[r, sl] = r0_v[r, sl] + r1_v[r, sl]
            return carry

        lax.fori_loop(0, SUB, _row, 0)
        pltpu.sync_copy(r0_v, y_hbm.at[pl.ds(base, SUB)])


@functools.lru_cache(maxsize=1)
def _combine_call_builder():
    return pl.kernel(
        _combine_kernel,
        mesh=plsc.VectorSubcoreMesh(core_axis_name="c", subcore_axis_name="s"),
        out_type=jax.ShapeDtypeStruct((S, D), jnp.float32),
        scratch_types=[
            pltpu.VMEM((SUB,), jnp.int32),
            pltpu.VMEM((SUB,), jnp.int32),
            pltpu.VMEM((SUB, D), jnp.float32),
            pltpu.VMEM((SUB, D), jnp.float32),
            pltpu.SemaphoreType.DMA,
            pltpu.SemaphoreType.DMA,
        ],
    )


def _combine_call(dpos_hbm, outs_hbm):
    return _combine_call_builder()(dpos_hbm, outs_hbm)


def kernel(x, Wr1, br1, Wg, We_in, We_out):
    bsz, length, d = x.shape
    x2d = x.reshape(S, D)
    gp1, gp2, dpos, te_row = _router_call(x2d, Wr1, br1.reshape(1, -1), Wg)
    dposf = dpos.reshape(NPAIR)
    te = te_row.reshape(NT)
    xs, gs = _scatter_call(dposf, x2d, gp1, gp2)
    outs = _gmm_call(te, xs, gs, We_in, We_out)
    y2 = _combine_call(dposf, outs)
    loss = jnp.zeros((), dtype=jnp.float32)
    return y2.reshape(bsz, length, d), loss


# speedup vs baseline: 1.0502x; 1.0011x over previous
import functools

import jax
import jax.numpy as jnp
from jax import lax
from jax.experimental import pallas as pl
from jax.experimental.pallas import tpu as pltpu
from jax.experimental.pallas import tpu_sc as plsc

E = 8
K = 2
D = 768
H = 768
S = 2048
NPAIR = S * K
TMC = 256
NP = NPAIR + E * TMC
NT = NPAIR // TMC + E
NC = 2
NS = 16
NW = NC * NS
BP = NPAIR // NW
TW = S // NW
SUB = 32
TS = 256
NB = S // TS


def _router_kernel(x_ref, wr1_ref, br1_ref, wg_ref,
                   gp1_ref, gp2_ref, dpos_ref, te_ref,
                   oh_scr):
    i = pl.program_id(0)

    def _router_step():
        xb = x_ref[...]
        h = lax.dot_general(xb, wr1_ref[...], (((1,), (1,)), ((), ())),
                            preferred_element_type=jnp.float32)
        h = jnp.tanh(h + br1_ref[...])
        logits = lax.dot_general(h, wg_ref[...], (((1,), (1,)), ((), ())),
                                 preferred_element_type=jnp.float32)
        lt = lax.transpose(logits, (1, 0))
        m = jnp.max(lt, axis=0, keepdims=True)
        p = jnp.exp(lt - m)
        p = p / jnp.sum(p, axis=0, keepdims=True)
        e_iota = lax.broadcasted_iota(jnp.int32, p.shape, 0)
        m1 = jnp.max(p, axis=0, keepdims=True)
        i1 = jnp.min(jnp.where(p == m1, e_iota, E), axis=0, keepdims=True)
        p_rest = jnp.where(e_iota == i1, -jnp.inf, p)
        m2 = jnp.max(p_rest, axis=0, keepdims=True)
        i2 = jnp.min(jnp.where(p_rest == m2, e_iota, E), axis=0, keepdims=True)
        denom = m1 + m2 + 1e-6
        g1 = m1 / denom
        g2 = m2 / denom
        pack = jnp.concatenate(
            [i1.astype(jnp.float32), i2.astype(jnp.float32), g1, g2,
             jnp.zeros((4, TS), jnp.float32)], axis=0)
        packT = lax.transpose(pack, (1, 0))
        i1c = packT[:, 0:1].astype(jnp.int32)
        i2c = packT[:, 1:2].astype(jnp.int32)
        e8 = lax.broadcasted_iota(jnp.int32, (TS, E), 1)
        oh_scr[pl.ds(i * TS, TS), :] = (e8 == i1c).astype(jnp.float32)
        oh_scr[pl.ds(S + i * TS, TS), :] = (e8 == i2c).astype(jnp.float32)
        gp1_ref[...] = jnp.broadcast_to(packT[:, 2:3], (TS, 128))
        gp2_ref[...] = jnp.broadcast_to(packT[:, 3:4], (TS, 128))

    _router_step()

    @pl.when(i == NB - 1)
    def _rank_step():
        r_i = lax.broadcasted_iota(jnp.int32, (256, 256), 0)
        c_i = lax.broadcasted_iota(jnp.int32, (256, 256), 1)
        tri = (c_i < r_i).astype(jnp.float32)

        nblk = NPAIR // 256
        t_rows = []
        rank_blocks = []
        for b in range(nblk):
            ohb = oh_scr[b * 256:(b + 1) * 256, :]
            rb = lax.dot_general(tri, ohb, (((1,), (0,)), ((), ())),
                                 preferred_element_type=jnp.float32)
            rank_blocks.append(rb)
            t_rows.append(jnp.sum(ohb, axis=0, keepdims=True))
        t_mat = jnp.concatenate(t_rows, axis=0)

        r16 = lax.broadcasted_iota(jnp.int32, (nblk, nblk), 0)
        c16 = lax.broadcasted_iota(jnp.int32, (nblk, nblk), 1)
        tri16 = (r16 < c16).astype(jnp.float32)
        bo = lax.dot_general(tri16, t_mat, (((0,), (0,)), ((), ())),
                             preferred_element_type=jnp.float32)

        counts = jnp.sum(t_mat, axis=0, keepdims=True)
        ci = counts.astype(jnp.int32)
        pci = ((ci + (TMC - 1)) // TMC) * TMC
        pcf = pci.astype(jnp.float32)
        r8 = lax.broadcasted_iota(jnp.int32, (E, E), 0)
        c8 = lax.broadcasted_iota(jnp.int32, (E, E), 1)
        tri8 = (r8 < c8).astype(jnp.float32)
        eo = lax.dot_general(pcf, tri8, (((1,), (0,)), ((), ())),
                             preferred_element_type=jnp.float32)

        dpos_parts = []
        for b in range(nblk):
            ohb = oh_scr[b * 256:(b + 1) * 256, :]
            base = rank_blocks[b] + bo[b:b + 1, :] + eo
            dpos_parts.append(jnp.sum(base * ohb, axis=1, keepdims=True))
        dpos = jnp.concatenate(dpos_parts, axis=0)
        dpos_ref[...] = dpos.astype(jnp.int32)

        ctoT = lax.transpose(eo * (1.0 / TMC), (1, 0))
        t_iota = lax.broadcasted_iota(jnp.int32, (E, NT), 1).astype(jnp.float32)
        ge = jnp.sum((t_iota >= ctoT).astype(jnp.int32), axis=0,
                     keepdims=True)
        te_ref[...] = ge - 1


def _router_call(x2d, Wr1, br1_2d, Wg, interpret=False):
    nmap = lambda i: (i, 0)
    zmap = lambda i: (0, 0)
    return pl.pallas_call(
        _router_kernel,
        grid=(NB,),
        in_specs=[
            pl.BlockSpec((TS, D), nmap),
            pl.BlockSpec((D, D), zmap),
            pl.BlockSpec((1, D), zmap),
            pl.BlockSpec((E, D), zmap),
        ],
        out_specs=[
            pl.BlockSpec((TS, 128), nmap),
            pl.BlockSpec((TS, 128), nmap),
            pl.BlockSpec((NPAIR, 1), zmap),
            pl.BlockSpec((1, NT), zmap),
        ],
        out_shape=[
            jax.ShapeDtypeStruct((S, 128), jnp.float32),
            jax.ShapeDtypeStruct((S, 128), jnp.float32),
            jax.ShapeDtypeStruct((NPAIR, 1), jnp.int32),
            jax.ShapeDtypeStruct((1, NT), jnp.int32),
        ],
        scratch_shapes=[pltpu.VMEM((NPAIR, E), jnp.float32)],
        compiler_params=pltpu.CompilerParams(
            dimension_semantics=("arbitrary",),
            vmem_limit_bytes=60 * 1024 * 1024,
        ),
        interpret=interpret,
    )(x2d, Wr1, br1_2d, Wg)


def _scatter_kernel(dpos_hbm, x_hbm, gp1_hbm, gp2_hbm, xs_hbm, gs_hbm,
                    idx0_v, idx1_v, rows_v, g0_v, g1_v, sem1, sem2):
    wid = lax.axis_index("s") * NC + lax.axis_index("c")
    base = wid * TW
    pltpu.sync_copy(dpos_hbm.at[pl.ds(base, TW)], idx0_v)
    pltpu.sync_copy(dpos_hbm.at[pl.ds(S + base, TW)], idx1_v)
    pltpu.sync_copy(x_hbm.at[pl.ds(base, TW)], rows_v)
    pltpu.sync_copy(gp1_hbm.at[pl.ds(base, TW)], g0_v)
    pltpu.sync_copy(gp2_hbm.at[pl.ds(base, TW)], g1_v)
    cp1 = pltpu.async_copy(rows_v, xs_hbm.at[idx0_v], sem1)
    cp2 = pltpu.async_copy(rows_v, xs_hbm.at[idx1_v], sem1)
    cp3 = pltpu.async_copy(g0_v, gs_hbm.at[idx0_v], sem2)
    cp4 = pltpu.async_copy(g1_v, gs_hbm.at[idx1_v], sem2)
    cp1.wait()
    cp2.wait()
    cp3.wait()
    cp4.wait()


@functools.lru_cache(maxsize=1)
def _scatter_call_builder():
    return pl.kernel(
        _scatter_kernel,
        mesh=plsc.VectorSubcoreMesh(core_axis_name="c", subcore_axis_name="s"),
        out_type=[
            jax.ShapeDtypeStruct((NP, D), jnp.float32),
            jax.ShapeDtypeStruct((NP, 128), jnp.float32),
        ],
        scratch_types=[
            pltpu.VMEM((TW,), jnp.int32),
            pltpu.VMEM((TW,), jnp.int32),
            pltpu.VMEM((TW, D), jnp.float32),
            pltpu.VMEM((TW, 128), jnp.float32),
            pltpu.VMEM((TW, 128), jnp.float32),
            pltpu.SemaphoreType.DMA,
            pltpu.SemaphoreType.DMA,
        ],
    )


def _scatter_call(dpos_hbm, x_hbm, gp1_hbm, gp2_hbm):
    return _scatter_call_builder()(dpos_hbm, x_hbm, gp1_hbm, gp2_hbm)


def _gmm_kernel(te_ref, xs_ref, gs_ref, win_ref, wout_ref, out_ref):
    e = te_ref[pl.program_id(0)]
    hh = lax.dot_general(xs_ref[...], win_ref[e], (((1,), (0,)), ((), ())),
                         preferred_element_type=jnp.float32)
    hh = jnp.maximum(hh, 0.0) * gs_ref[:, 0:1]
    oo = lax.dot_general(hh, wout_ref[e], (((1,), (0,)), ((), ())),
                         preferred_element_type=jnp.float32)
    out_ref[...] = oo


def _gmm_call(te, xs, gs, We_in, We_out, interpret=False):
    return pl.pallas_call(
        _gmm_kernel,
        grid_spec=pltpu.PrefetchScalarGridSpec(
            num_scalar_prefetch=1,
            grid=(NT,),
            in_specs=[
                pl.BlockSpec((TMC, D), lambda t, te_ref: (t, 0)),
                pl.BlockSpec((TMC, 128), lambda t, te_ref: (t, 0)),
                pl.BlockSpec((E, D, H), lambda t, te_ref: (0, 0, 0)),
                pl.BlockSpec((E, H, D), lambda t, te_ref: (0, 0, 0)),
            ],
            out_specs=pl.BlockSpec((TMC, D), lambda t, te_ref: (t, 0)),
        ),
        out_shape=jax.ShapeDtypeStruct((NP, D), jnp.float32),
        compiler_params=pltpu.CompilerParams(
            dimension_semantics=("arbitrary",),
            vmem_limit_bytes=60 * 1024 * 1024,
        ),
        interpret=interpret,
    )(te, xs, gs, We_in, We_out)


def _combine_kernel(dpos_hbm, outs_hbm, y_hbm,
                    p0_v, p1_v, r0_v, r1_v, sem0, sem1):
    wid = lax.axis_index("s") * NC + lax.axis_index("c")
    for sub in range(TW // SUB):
        base = wid * TW + sub * SUB
        pltpu.sync_copy(dpos_hbm.at[pl.ds(base, SUB)], p0_v)
        pltpu.sync_copy(dpos_hbm.at[pl.ds(S + base, SUB)], p1_v)
        cp0 = pltpu.async_copy(outs_hbm.at[p0_v], r0_v, sem0)
        cp1 = pltpu.async_copy(outs_hbm.at[p1_v], r1_v, sem1)
        cp0.wait()
        cp1.wait()

        def _row(r, carry):
            for j in range(D // 16):
                sl = pl.ds(j * 16, 16)
                r0_v[r, sl] = r0_v[r, sl] + r1_v[r, sl]
            return carry

        lax.fori_loop(0, SUB, _row, 0)
        pltpu.sync_copy(r0_v, y_hbm.at[pl.ds(base, SUB)])


@functools.lru_cache(maxsize=1)
def _combine_call_builder():
    return pl.kernel(
        _combine_kernel,
        mesh=plsc.VectorSubcoreMesh(core_axis_name="c", subcore_axis_name="s"),
        out_type=jax.ShapeDtypeStruct((S, D), jnp.float32),
        scratch_types=[
            pltpu.VMEM((SUB,), jnp.int32),
            pltpu.VMEM((SUB,), jnp.int32),
            pltpu.VMEM((SUB, D), jnp.float32),
            pltpu.VMEM((SUB, D), jnp.float32),
            pltpu.SemaphoreType.DMA,
            pltpu.SemaphoreType.DMA,
        ],
    )


def _combine_call(dpos_hbm, outs_hbm):
    return _combine_call_builder()(dpos_hbm, outs_hbm)


def kernel(x, Wr1, br1, Wg, We_in, We_out):
    bsz, length, d = x.shape
    x2d = x.reshape(S, D)
    gp1, gp2, dpos, te_row = _router_call(x2d, Wr1, br1.reshape(1, -1), Wg)
    dposf = dpos.reshape(NPAIR)
    te = te_row.reshape(NT)
    xs, gs = _scatter_call(dposf, x2d, gp1, gp2)
    outs = _gmm_call(te, xs, gs, We_in, We_out)
    y2 = _combine_call(dposf, outs)
    loss = jnp.zeros((), dtype=jnp.float32)
    return y2.reshape(bsz, length, d), loss
